# baseline (device time: 21174 ns/iter reference)
import jax
import jax.numpy as jnp
from jax import lax
from jax.experimental import pallas as pl
from jax.experimental.pallas import tpu as pltpu

N_DEV = 4
N_LAYERS = 3


def kernel(x, Win0, Wout0, Win1, Wout1, Win2, Wout2):
    b, d = x.shape
    out_rows = b // N_DEV

    def body(x_ref, win0, wout0, win1, wout1, win2, wout2,
             out_ref, send_buf, comm_ref, acc_ref, send_sems, recv_sems):
        my = lax.axis_index("i")

        barrier = pltpu.get_barrier_semaphore()
        for off in range(1, N_DEV):
            peer = (my + off) % N_DEV
            pl.semaphore_signal(
                barrier, inc=1,
                device_id=(peer,), device_id_type=pl.DeviceIdType.MESH,
            )
        pl.semaphore_wait(barrier, N_DEV - 1)

        layers = [(win0, wout0), (win1, wout1), (win2, wout2)]
        xv = x_ref[:, :]
        for l, (win, wout) in enumerate(layers):
            h = jnp.maximum(
                jnp.dot(xv, win[:, :], preferred_element_type=jnp.float32), 0.0
            )
            partial = jnp.dot(h, wout[:, :], preferred_element_type=jnp.float32)
            send_buf[l] = partial

            sends = []
            for off in range(1, N_DEV):
                peer = (my + off) % N_DEV
                rdma = pltpu.make_async_remote_copy(
                    src_ref=send_buf.at[l],
                    dst_ref=comm_ref.at[l, off - 1],
                    send_sem=send_sems.at[l, off - 1],
                    recv_sem=recv_sems.at[l, off - 1],
                    device_id=(peer,),
                    device_id_type=pl.DeviceIdType.MESH,
                )
                rdma.start()
                sends.append(rdma)
            for rdma in sends:
                rdma.wait_send()

            total = partial
            for off in range(1, N_DEV):
                peer = (my + off) % N_DEV
                recv = pltpu.make_async_remote_copy(
                    src_ref=send_buf.at[l],
                    dst_ref=comm_ref.at[l, off - 1],
                    send_sem=send_sems.at[l, off - 1],
                    recv_sem=recv_sems.at[l, off - 1],
                    device_id=(peer,),
                    device_id_type=pl.DeviceIdType.MESH,
                )
                recv.wait_recv()
                total = total + comm_ref[l, off - 1]
            xv = total

        acc_ref[:, :] = xv
        out_ref[:, :] = acc_ref[pl.ds(my * out_rows, out_rows), :]

    return pl.pallas_call(
        body,
        out_shape=jax.ShapeDtypeStruct((out_rows, d), jnp.float32),
        in_specs=[pl.BlockSpec(memory_space=pltpu.VMEM)] * 7,
        out_specs=pl.BlockSpec(memory_space=pltpu.VMEM),
        scratch_shapes=[
            pltpu.VMEM((N_LAYERS, b, d), jnp.float32),
            pltpu.VMEM((N_LAYERS, N_DEV - 1, b, d), jnp.float32),
            pltpu.VMEM((b, d), jnp.float32),
            pltpu.SemaphoreType.DMA((N_LAYERS, N_DEV - 1)),
            pltpu.SemaphoreType.DMA((N_LAYERS, N_DEV - 1)),
        ],
        compiler_params=pltpu.CompilerParams(collective_id=0),
    )(x, Win0, Wout0, Win1, Wout1, Win2, Wout2)


# device time: 19985 ns/iter; 1.0595x vs baseline; 1.0595x over previous
import jax
import jax.numpy as jnp
from jax import lax
from jax.experimental import pallas as pl
from jax.experimental.pallas import tpu as pltpu

N_DEV = 4
N_LAYERS = 3


def kernel(x, Win0, Wout0, Win1, Wout1, Win2, Wout2):
    b, d = x.shape
    out_rows = b // N_DEV

    def body(x_ref, win0, wout0, win1, wout1, win2, wout2,
             out_ref, send_buf, comm_ref, rs_ref, send_sems, recv_sems):
        my = lax.axis_index("i")

        barrier = pltpu.get_barrier_semaphore()
        for off in range(1, N_DEV):
            peer = (my + off) % N_DEV
            pl.semaphore_signal(
                barrier, inc=1,
                device_id=(peer,), device_id_type=pl.DeviceIdType.MESH,
            )
        pl.semaphore_wait(barrier, N_DEV - 1)

        layers = [(win0, wout0), (win1, wout1), (win2, wout2)]
        pending = []
        xv = x_ref[:, :]
        for l, (win, wout) in enumerate(layers):
            h = jnp.maximum(
                jnp.dot(xv, win[:, :], preferred_element_type=jnp.float32), 0.0
            )
            partial = jnp.dot(h, wout[:, :], preferred_element_type=jnp.float32)
            send_buf[l] = partial

            last = l == N_LAYERS - 1
            for off in (2, 1, 3):
                peer = (my + off) % N_DEV
                if last:
                    src = send_buf.at[l, pl.ds(peer * out_rows, out_rows)]
                    dst = rs_ref.at[off - 1]
                else:
                    src = send_buf.at[l]
                    dst = comm_ref.at[l, off - 1]
                rdma = pltpu.make_async_remote_copy(
                    src_ref=src,
                    dst_ref=dst,
                    send_sem=send_sems.at[l, off - 1],
                    recv_sem=recv_sems.at[l, off - 1],
                    device_id=(peer,),
                    device_id_type=pl.DeviceIdType.MESH,
                )
                rdma.start()
                pending.append(rdma)

            if last:
                acc = send_buf[l, pl.ds(my * out_rows, out_rows)]
            else:
                acc = partial
            for off in (1, 2, 3):
                if last:
                    src = send_buf.at[l, pl.ds(my * out_rows, out_rows)]
                    dst = rs_ref.at[off - 1]
                else:
                    src = send_buf.at[l]
                    dst = comm_ref.at[l, off - 1]
                recv = pltpu.make_async_remote_copy(
                    src_ref=src,
                    dst_ref=dst,
                    send_sem=send_sems.at[l, off - 1],
                    recv_sem=recv_sems.at[l, off - 1],
                    device_id=((my + off) % N_DEV,),
                    device_id_type=pl.DeviceIdType.MESH,
                )
                recv.wait_recv()
                if last:
                    acc = acc + rs_ref[off - 1]
                else:
                    acc = acc + comm_ref[l, off - 1]
            if last:
                out_ref[:, :] = acc
            else:
                xv = acc

        for rdma in pending:
            rdma.wait_send()

    return pl.pallas_call(
        body,
        out_shape=jax.ShapeDtypeStruct((out_rows, d), jnp.float32),
        in_specs=[pl.BlockSpec(memory_space=pltpu.VMEM)] * 7,
        out_specs=pl.BlockSpec(memory_space=pltpu.VMEM),
        scratch_shapes=[
            pltpu.VMEM((N_LAYERS, b, d), jnp.float32),
            pltpu.VMEM((N_LAYERS - 1, N_DEV - 1, b, d), jnp.float32),
            pltpu.VMEM((N_DEV - 1, out_rows, d), jnp.float32),
            pltpu.SemaphoreType.DMA((N_LAYERS, N_DEV - 1)),
            pltpu.SemaphoreType.DMA((N_LAYERS, N_DEV - 1)),
        ],
        compiler_params=pltpu.CompilerParams(collective_id=0),
    )(x, Win0, Wout0, Win1, Wout1, Win2, Wout2)


# device time: 19284 ns/iter; 1.0980x vs baseline; 1.0364x over previous
import jax
import jax.numpy as jnp
from jax import lax
from jax.experimental import pallas as pl
from jax.experimental.pallas import tpu as pltpu

N_DEV = 4
N_LAYERS = 3


def kernel(x, Win0, Wout0, Win1, Wout1, Win2, Wout2):
    b, d = x.shape
    out_rows = b // N_DEV

    def body(x_ref, win0, wout0, win1, wout1, win2, wout2,
             out_ref, send_buf, comm_ref, rs_ref, send_sems, recv_sems):
        my = lax.axis_index("i")

        layers = [(win0, wout0), (win1, wout1), (win2, wout2)]

        xv = x_ref[:, :]
        h = jnp.maximum(
            jnp.dot(xv, layers[0][0][:, :], preferred_element_type=jnp.float32), 0.0
        )
        partial = jnp.dot(h, layers[0][1][:, :], preferred_element_type=jnp.float32)
        send_buf[0] = partial

        barrier = pltpu.get_barrier_semaphore()
        for nbr in [(my + 1) % N_DEV, (my - 1) % N_DEV]:
            pl.semaphore_signal(
                barrier, inc=1,
                device_id=(nbr,), device_id_type=pl.DeviceIdType.MESH,
            )
        pl.semaphore_wait(barrier, 2)

        pending = []
        for l, (win, wout) in enumerate(layers):
            last = l == N_LAYERS - 1
            for off in (2, 1, 3):
                peer = (my + off) % N_DEV
                if last:
                    src = send_buf.at[l, pl.ds(peer * out_rows, out_rows)]
                    dst = rs_ref.at[off - 1]
                else:
                    src = send_buf.at[l]
                    dst = comm_ref.at[l, off - 1]
                rdma = pltpu.make_async_remote_copy(
                    src_ref=src,
                    dst_ref=dst,
                    send_sem=send_sems.at[l, off - 1],
                    recv_sem=recv_sems.at[l, off - 1],
                    device_id=(peer,),
                    device_id_type=pl.DeviceIdType.MESH,
                )
                rdma.start()
                pending.append(rdma)

            if last:
                acc = send_buf[l, pl.ds(my * out_rows, out_rows)]
            else:
                acc = partial
            for off in (1, 3, 2):
                if last:
                    src = send_buf.at[l, pl.ds(my * out_rows, out_rows)]
                    dst = rs_ref.at[off - 1]
                else:
                    src = send_buf.at[l]
                    dst = comm_ref.at[l, off - 1]
                recv = pltpu.make_async_remote_copy(
                    src_ref=src,
                    dst_ref=dst,
                    send_sem=send_sems.at[l, off - 1],
                    recv_sem=recv_sems.at[l, off - 1],
                    device_id=((my + off) % N_DEV,),
                    device_id_type=pl.DeviceIdType.MESH,
                )
                recv.wait_recv()
                if last:
                    acc = acc + rs_ref[off - 1]
                else:
                    acc = acc + comm_ref[l, off - 1]
            if last:
                out_ref[:, :] = acc
            else:
                h = jnp.maximum(
                    jnp.dot(acc, layers[l + 1][0][:, :],
                            preferred_element_type=jnp.float32),
                    0.0,
                )
                partial = jnp.dot(
                    h, layers[l + 1][1][:, :], preferred_element_type=jnp.float32
                )
                send_buf[l + 1] = partial

        for rdma in pending:
            rdma.wait_send()

    return pl.pallas_call(
        body,
        out_shape=jax.ShapeDtypeStruct((out_rows, d), jnp.float32),
        in_specs=[pl.BlockSpec(memory_space=pltpu.VMEM)] * 7,
        out_specs=pl.BlockSpec(memory_space=pltpu.VMEM),
        scratch_shapes=[
            pltpu.VMEM((N_LAYERS, b, d), jnp.float32),
            pltpu.VMEM((N_LAYERS - 1, N_DEV - 1, b, d), jnp.float32),
            pltpu.VMEM((N_DEV - 1, out_rows, d), jnp.float32),
            pltpu.SemaphoreType.DMA((N_LAYERS, N_DEV - 1)),
            pltpu.SemaphoreType.DMA((N_LAYERS, N_DEV - 1)),
        ],
        compiler_params=pltpu.CompilerParams(collective_id=0),
    )(x, Win0, Wout0, Win1, Wout1, Win2, Wout2)


# device time: 18155 ns/iter; 1.1663x vs baseline; 1.0622x over previous
import jax
import jax.numpy as jnp
from jax import lax
from jax.experimental import pallas as pl
from jax.experimental.pallas import tpu as pltpu

N_DEV = 4
N_LAYERS = 3


def kernel(x, Win0, Wout0, Win1, Wout1, Win2, Wout2):
    b, d = x.shape
    out_rows = b // N_DEV

    def body(x_ref, win0, wout0, win1, wout1, win2, wout2,
             out_ref, send_buf, comm_ref, rs_ref, own_ref, send_sems, recv_sems):
        my = lax.axis_index("i")

        layers = [(win0, wout0), (win1, wout1), (win2, wout2)]

        xv = x_ref[:, :]
        h = jnp.maximum(
            jnp.dot(xv, layers[0][0][:, :], preferred_element_type=jnp.float32), 0.0
        )
        partial = jnp.dot(h, layers[0][1][:, :], preferred_element_type=jnp.float32)
        send_buf[0] = partial.astype(jnp.bfloat16)

        barrier = pltpu.get_barrier_semaphore()
        for nbr in [(my + 1) % N_DEV, (my - 1) % N_DEV]:
            pl.semaphore_signal(
                barrier, inc=1,
                device_id=(nbr,), device_id_type=pl.DeviceIdType.MESH,
            )
        pl.semaphore_wait(barrier, 2)

        pending = []
        for l in range(N_LAYERS):
            last = l == N_LAYERS - 1
            if last:
                own_ref[:, :] = partial
            for off in (2, 1, 3):
                peer = (my + off) % N_DEV
                if last:
                    src = send_buf.at[l, pl.ds(peer * out_rows, out_rows)]
                    dst = rs_ref.at[off - 1]
                else:
                    src = send_buf.at[l]
                    dst = comm_ref.at[l, off - 1]
                rdma = pltpu.make_async_remote_copy(
                    src_ref=src,
                    dst_ref=dst,
                    send_sem=send_sems.at[l, off - 1],
                    recv_sem=recv_sems.at[l, off - 1],
                    device_id=(peer,),
                    device_id_type=pl.DeviceIdType.MESH,
                )
                rdma.start()
                pending.append(rdma)

            if last:
                acc = own_ref[pl.ds(my * out_rows, out_rows), :]
            else:
                acc = partial
            for off in (1, 3, 2):
                if last:
                    src = send_buf.at[l, pl.ds(my * out_rows, out_rows)]
                    dst = rs_ref.at[off - 1]
                else:
                    src = send_buf.at[l]
                    dst = comm_ref.at[l, off - 1]
                recv = pltpu.make_async_remote_copy(
                    src_ref=src,
                    dst_ref=dst,
                    send_sem=send_sems.at[l, off - 1],
                    recv_sem=recv_sems.at[l, off - 1],
                    device_id=((my + off) % N_DEV,),
                    device_id_type=pl.DeviceIdType.MESH,
                )
                recv.wait_recv()
                if last:
                    acc = acc + rs_ref[off - 1].astype(jnp.float32)
                else:
                    acc = acc + comm_ref[l, off - 1].astype(jnp.float32)
            if last:
                out_ref[:, :] = acc
            else:
                h = jnp.maximum(
                    jnp.dot(acc, layers[l + 1][0][:, :],
                            preferred_element_type=jnp.float32),
                    0.0,
                )
                partial = jnp.dot(
                    h, layers[l + 1][1][:, :], preferred_element_type=jnp.float32
                )
                send_buf[l + 1] = partial.astype(jnp.bfloat16)

        for rdma in pending:
            rdma.wait_send()

    return pl.pallas_call(
        body,
        out_shape=jax.ShapeDtypeStruct((out_rows, d), jnp.float32),
        in_specs=[pl.BlockSpec(memory_space=pltpu.VMEM)] * 7,
        out_specs=pl.BlockSpec(memory_space=pltpu.VMEM),
        scratch_shapes=[
            pltpu.VMEM((N_LAYERS, b, d), jnp.bfloat16),
            pltpu.VMEM((N_LAYERS - 1, N_DEV - 1, b, d), jnp.bfloat16),
            pltpu.VMEM((N_DEV - 1, out_rows, d), jnp.bfloat16),
            pltpu.VMEM((b, d), jnp.float32),
            pltpu.SemaphoreType.DMA((N_LAYERS, N_DEV - 1)),
            pltpu.SemaphoreType.DMA((N_LAYERS, N_DEV - 1)),
        ],
        compiler_params=pltpu.CompilerParams(collective_id=0),
    )(x, Win0, Wout0, Win1, Wout1, Win2, Wout2)
